# baseline (speedup 1.0000x reference)
import functools

import jax
import jax.numpy as jnp
from jax import lax
from jax.experimental import pallas as pl
from jax.experimental.pallas import tpu as pltpu
from jax.experimental.pallas import tpu_sc as plsc

TOP_K = 8
NUM_EXPERTS = 64
HIDDEN = 768
TOKENS = 32768
BT = 4096
NW = 32
TPW = TOKENS // NW


def _gate_block(hs_ref, gw_ref, lg_ref):
    lt = jax.lax.dot_general(
        gw_ref[...], hs_ref[...],
        dimension_numbers=(((1,), (1,)), ((), ())),
        preferred_element_type=jnp.float32,
    )
    lg_ref[...] = lt.T


def _gate_logits(hidden_states, gate_weight):
    return pl.pallas_call(
        _gate_block,
        grid=(TOKENS // BT,),
        in_specs=[
            pl.BlockSpec((BT, HIDDEN), lambda t: (t, 0)),
            pl.BlockSpec((NUM_EXPERTS, HIDDEN), lambda t: (0, 0)),
        ],
        out_specs=pl.BlockSpec((BT, NUM_EXPERTS), lambda t: (t, 0)),
        out_shape=jax.ShapeDtypeStruct((TOKENS, NUM_EXPERTS), jnp.float32),
        compiler_params=pltpu.CompilerParams(
            dimension_semantics=("arbitrary",),
        ),
    )(hidden_states, gate_weight)


@functools.partial(
    pl.kernel,
    mesh=plsc.VectorSubcoreMesh(core_axis_name="c", subcore_axis_name="s"),
    out_type=[
        jax.ShapeDtypeStruct((TOKENS * TOP_K,), jnp.float32),
        jax.ShapeDtypeStruct((TOKENS * TOP_K,), jnp.int32),
    ],
    scratch_types=[
        pltpu.VMEM((TPW * NUM_EXPERTS,), jnp.float32),
        pltpu.VMEM((TPW * TOP_K + 8,), jnp.float32),
        pltpu.VMEM((TPW * TOP_K + 8,), jnp.int32),
    ],
    compiler_params=pltpu.CompilerParams(needs_layout_passes=False),
)
def _sc_topk(lg_hbm, w_hbm, i_hbm, slab, wbuf, ibuf):
    wid = lax.axis_index("s") * 2 + lax.axis_index("c")
    base = wid * TPW
    pltpu.sync_copy(lg_hbm.at[pl.ds(base * NUM_EXPERTS, TPW * NUM_EXPERTS)],
                    slab)
    lanes = lax.iota(jnp.int32, 16)
    v0 = lanes
    v1 = lanes + 16
    v2 = lanes + 32
    v3 = lanes + 48
    mask8 = lanes < TOP_K

    @plsc.parallel_loop(0, TPW, 1, unroll=8)
    def body(t):
        off = t * NUM_EXPERTS
        k0 = slab[pl.ds(off, 16)]
        k1 = slab[pl.ds(off + 16, 16)]
        k2 = slab[pl.ds(off + 32, 16)]
        k3 = slab[pl.ds(off + 48, 16)]
        s0k, s0v = plsc.sort_key_val(k0, v0, descending=True)
        s1k, s1v = plsc.sort_key_val(k1, v1, descending=False)
        s2k, s2v = plsc.sort_key_val(k2, v2, descending=True)
        s3k, s3v = plsc.sort_key_val(k3, v3, descending=False)
        c0 = s0k >= s1k
        h0k = jnp.where(c0, s0k, s1k)
        h0v = jnp.where(c0, s0v, s1v)
        c1 = s2k >= s3k
        h1k = jnp.where(c1, s2k, s3k)
        h1v = jnp.where(c1, s2v, s3v)
        m0k, m0v = plsc.sort_key_val(h0k, h0v, descending=True)
        m1k, m1v = plsc.sort_key_val(h1k, h1v, descending=False)
        cf = m0k >= m1k
        fk = jnp.where(cf, m0k, m1k)
        fv = jnp.where(cf, m0v, m1v)
        tk, tv = plsc.sort_key_val(fk, fv, descending=True)
        e = jnp.exp(tk - tk[0])
        e8 = jnp.where(mask8, e, jnp.float32(0.0))
        w = e8 / jnp.sum(e8)
        plsc.store_compressed(wbuf.at[pl.ds(t * TOP_K, 16)], w, mask=mask8)
        plsc.store_compressed(ibuf.at[pl.ds(t * TOP_K, 16)], tv, mask=mask8)

    pltpu.sync_copy(wbuf.at[pl.ds(0, TPW * TOP_K)],
                    w_hbm.at[pl.ds(base * TOP_K, TPW * TOP_K)])
    pltpu.sync_copy(ibuf.at[pl.ds(0, TPW * TOP_K)],
                    i_hbm.at[pl.ds(base * TOP_K, TPW * TOP_K)])


@jax.jit
def kernel(hidden_states, gate_weight):
    lg = _gate_logits(hidden_states, gate_weight)
    w_flat, i_flat = _sc_topk(lg.reshape(-1))
    return (w_flat.reshape(TOKENS, TOP_K), i_flat.reshape(TOKENS, TOP_K))


# speedup vs baseline: 3.1866x; 3.1866x over previous
import jax
import jax.numpy as jnp
from jax.experimental import pallas as pl
from jax.experimental.pallas import tpu as pltpu

TOP_K = 8
NUM_EXPERTS = 64
HIDDEN = 768
TOKENS = 32768
BT = 4096


def _router_block(hs_ref, gw_ref, w_ref, i_ref):
    lt = jax.lax.dot_general(
        gw_ref[...], hs_ref[...],
        dimension_numbers=(((1,), (1,)), ((), ())),
        preferred_element_type=jnp.float32,
    )
    row = jax.lax.broadcasted_iota(jnp.int32, (NUM_EXPERTS, BT), 0)
    x = lt
    neg_inf = jnp.float32(-jnp.inf)
    vals = []
    idxs = []
    for _ in range(TOP_K):
        m = jnp.max(x, axis=0, keepdims=True)
        hit = x >= m
        a = jnp.min(jnp.where(hit, row, NUM_EXPERTS), axis=0,
                    keepdims=True)
        vals.append(m)
        idxs.append(a)
        x = jnp.where(row == a, neg_inf, x)
    v = jnp.concatenate(vals, axis=0)
    e = jnp.exp(v - v[0:1, :])
    w_ref[...] = e / jnp.sum(e, axis=0, keepdims=True)
    i_ref[...] = jnp.concatenate(idxs, axis=0)


@jax.jit
def kernel(hidden_states, gate_weight):
    grid = (TOKENS // BT,)
    w, i = pl.pallas_call(
        _router_block,
        grid=grid,
        in_specs=[
            pl.BlockSpec((BT, HIDDEN), lambda t: (t, 0)),
            pl.BlockSpec((NUM_EXPERTS, HIDDEN), lambda t: (0, 0)),
        ],
        out_specs=[
            pl.BlockSpec((TOP_K, BT), lambda t: (0, t)),
            pl.BlockSpec((TOP_K, BT), lambda t: (0, t)),
        ],
        out_shape=[
            jax.ShapeDtypeStruct((TOP_K, TOKENS), jnp.float32),
            jax.ShapeDtypeStruct((TOP_K, TOKENS), jnp.int32),
        ],
        compiler_params=pltpu.CompilerParams(
            dimension_semantics=("parallel",),
        ),
    )(hidden_states, gate_weight)
    return (w.T, i.T)
